# baseline (speedup 1.0000x reference)
import jax
import jax.numpy as jnp
from jax.experimental import pallas as pl
from jax.experimental.pallas import tpu as pltpu

_N = 65536
_TILE = 8192
_GRID = _N // _TILE

def _probe_kernel(x_ref, c_ref, idx_ref, part_ref):
    idx_ref[0, 0, :] = jnp.broadcast_to(x_ref[0, 0].astype(jnp.int32), (_TILE,))
    part_ref[0, 0, 0] = c_ref[0, 0]

def kernel(x, cluster):
    idx2d, partials = pl.pallas_call(
        _probe_kernel,
        grid=(_GRID,),
        in_specs=[
            pl.BlockSpec((_TILE // 4, 128), lambda i: (i, 0)),
            pl.BlockSpec((512, 32), lambda i: (0, 0)),
        ],
        out_specs=[
            pl.BlockSpec((1, 1, _TILE), lambda i: (i, 0, 0)),
            pl.BlockSpec((1, 1, 1), lambda i: (i, 0, 0), memory_space=pltpu.SMEM),
        ],
        out_shape=[
            jax.ShapeDtypeStruct((_GRID, 1, _TILE), jnp.int32),
            jax.ShapeDtypeStruct((_GRID, 1, 1), jnp.float32),
        ],
        compiler_params=pltpu.CompilerParams(
            dimension_semantics=("arbitrary",)),
    )(x.reshape(_N // 4, 128), cluster)
    return (idx2d.reshape(_N), jnp.sum(partials))


# speedup vs baseline: 1.8810x; 1.8810x over previous
import jax
import jax.numpy as jnp
from jax.experimental import pallas as pl
from jax.experimental.pallas import tpu as pltpu

_N = 65536
_NDIM = 32
_K = 512
_TILE = 8192
_GRID = _N // _TILE


def _assign_kernel(x_ref, c_ref, idx_ref, part_ref):
    xt = x_ref[...]
    c = c_ref[...]
    xsq = xt * xt
    x2sum = jnp.sum(xsq)
    ch2 = 0.5 * jnp.sum(c * c, axis=1, keepdims=True)
    s = jax.lax.dot_general(
        c, xt, (((1,), (0,)), ((), ())),
        preferred_element_type=jnp.float32)
    ch = 8
    acc_v = ch2[0:ch] - s[0:ch]
    acc_slab = jnp.zeros((ch, _TILE), jnp.int32)
    for k in range(1, _K // ch):
        blk = ch2[ch * k:ch * (k + 1)] - s[ch * k:ch * (k + 1)]
        lt = blk < acc_v
        acc_v = jnp.minimum(blk, acc_v)
        acc_slab = jnp.where(lt, jnp.int32(k), acc_slab)
    v = acc_v
    i = acc_slab * ch + jax.lax.broadcasted_iota(jnp.int32, (ch, _TILE), 0)
    h = ch
    while h > 1:
        h //= 2
        va, vb = v[0:h], v[h:2 * h]
        ia, ib = i[0:h], i[h:2 * h]
        better = (vb < va) | ((vb == va) & (ib < ia))
        v = jnp.where(better, vb, va)
        i = jnp.where(better, ib, ia)
    idx_ref[0, 0, :] = i[0]
    part_ref[0, 0, 0] = x2sum + 2.0 * jnp.sum(v)


def kernel(x, cluster):
    idx2d, partials = pl.pallas_call(
        _assign_kernel,
        grid=(_GRID,),
        in_specs=[
            pl.BlockSpec((_NDIM, _TILE), lambda i: (0, i)),
            pl.BlockSpec((_K, _NDIM), lambda i: (0, 0)),
        ],
        out_specs=[
            pl.BlockSpec((1, 1, _TILE), lambda i: (i, 0, 0)),
            pl.BlockSpec((1, 1, 1), lambda i: (i, 0, 0), memory_space=pltpu.SMEM),
        ],
        out_shape=[
            jax.ShapeDtypeStruct((_GRID, 1, _TILE), jnp.int32),
            jax.ShapeDtypeStruct((_GRID, 1, 1), jnp.float32),
        ],
        compiler_params=pltpu.CompilerParams(
            dimension_semantics=("arbitrary",)),
    )(x.T, cluster)
    idx = idx2d.reshape(_N)
    loss = jnp.sum(partials) / jnp.float32(_N)
    return (idx, loss)


# speedup vs baseline: 1.8903x; 1.0049x over previous
import jax
import jax.numpy as jnp
from jax.experimental import pallas as pl
from jax.experimental.pallas import tpu as pltpu

_N = 65536
_NDIM = 32
_K = 512
_TILE = 16384
_GRID = _N // _TILE


def _assign_kernel(x_ref, c_ref, idx_ref, part_ref):
    xt = x_ref[...]
    c = c_ref[...]
    xsq = xt * xt
    x2sum = jnp.sum(xsq)
    ch2 = 0.5 * jnp.sum(c * c, axis=1, keepdims=True)
    s = jax.lax.dot_general(
        c, xt, (((1,), (0,)), ((), ())),
        preferred_element_type=jnp.float32)
    ch = 8
    acc_v = ch2[0:ch] - s[0:ch]
    acc_slab = jnp.zeros((ch, _TILE), jnp.int32)
    for k in range(1, _K // ch):
        blk = ch2[ch * k:ch * (k + 1)] - s[ch * k:ch * (k + 1)]
        lt = blk < acc_v
        acc_v = jnp.minimum(blk, acc_v)
        acc_slab = jnp.where(lt, jnp.int32(k), acc_slab)
    v = acc_v
    i = acc_slab * ch + jax.lax.broadcasted_iota(jnp.int32, (ch, _TILE), 0)
    h = ch
    while h > 1:
        h //= 2
        va, vb = v[0:h], v[h:2 * h]
        ia, ib = i[0:h], i[h:2 * h]
        better = (vb < va) | ((vb == va) & (ib < ia))
        v = jnp.where(better, vb, va)
        i = jnp.where(better, ib, ia)
    idx_ref[0, 0, :] = i[0]
    part_ref[0, 0, 0] = x2sum + 2.0 * jnp.sum(v)


def kernel(x, cluster):
    idx2d, partials = pl.pallas_call(
        _assign_kernel,
        grid=(_GRID,),
        in_specs=[
            pl.BlockSpec((_NDIM, _TILE), lambda i: (0, i)),
            pl.BlockSpec((_K, _NDIM), lambda i: (0, 0)),
        ],
        out_specs=[
            pl.BlockSpec((1, 1, _TILE), lambda i: (i, 0, 0)),
            pl.BlockSpec((1, 1, 1), lambda i: (i, 0, 0), memory_space=pltpu.SMEM),
        ],
        out_shape=[
            jax.ShapeDtypeStruct((_GRID, 1, _TILE), jnp.int32),
            jax.ShapeDtypeStruct((_GRID, 1, 1), jnp.float32),
        ],
        compiler_params=pltpu.CompilerParams(
            dimension_semantics=("arbitrary",)),
    )(x.T, cluster)
    idx = idx2d.reshape(_N)
    loss = jnp.sum(partials) / jnp.float32(_N)
    return (idx, loss)
